# baseline (speedup 1.0000x reference)
import functools
import math

import jax
import jax.numpy as jnp
from jax import lax
from jax.experimental import pallas as pl
from jax.experimental.pallas import tpu as pltpu
from jax.experimental.pallas import tpu_sc as plsc

N_NODES = 10000
N_EDGES = 160000
D_EDGE = 64
HID = 64
SPH = 32
K = 9
FEAT = K * SPH

NC = 2
NS = 16
NW = NC * NS

CH = 128
NCHUNK = N_EDGES // CH

HALF = N_NODES // 2
ACC_ROWS = 5120
ROWS_PER_TILE = ACC_ROWS // NS

BLK = 640
GRID = N_EDGES // BLK

@functools.cache
def _mesh():
  return plsc.VectorSubcoreMesh(
      core_axis_name="c", subcore_axis_name="s", num_cores=NC, num_subcores=NS
  )


CHA = 640
NCHUNK_A = N_EDGES // CHA


def _gather_species(species, senders, receivers):
  @functools.partial(
      pl.kernel,
      out_type=[
          jax.ShapeDtypeStruct((N_EDGES,), jnp.int32),
          jax.ShapeDtypeStruct((N_EDGES,), jnp.int32),
      ],
      mesh=_mesh(),
      compiler_params=pltpu.CompilerParams(
          needs_layout_passes=False, use_tc_tiling_on_sc=True),
      scratch_types=[
          pltpu.VMEM((N_NODES,), jnp.int32),
          pltpu.VMEM((CHA,), jnp.int32),
          pltpu.VMEM((CHA,), jnp.int32),
          pltpu.VMEM((CHA,), jnp.int32),
          pltpu.VMEM((CHA,), jnp.int32),
      ],
  )
  def k(species_hbm, senders_hbm, receivers_hbm, ssp_hbm, rsp_hbm,
        spec_v, sidx_v, ridx_v, ssp_v, rsp_v):
    wid = lax.axis_index("s") * NC + lax.axis_index("c")
    pltpu.sync_copy(species_hbm, spec_v)

    def body(i, carry):
      chunk = i * NW + wid

      @pl.when(chunk < NCHUNK_A)
      def _():
        base = chunk * CHA
        pltpu.sync_copy(senders_hbm.at[pl.ds(base, CHA)], sidx_v)
        pltpu.sync_copy(receivers_hbm.at[pl.ds(base, CHA)], ridx_v)
        for g in range(CHA // 16):
          sl = pl.ds(g * 16, 16)
          ssp_v[sl] = plsc.load_gather(spec_v, [sidx_v[sl]])
          rsp_v[sl] = plsc.load_gather(spec_v, [ridx_v[sl]])
        pltpu.sync_copy(ssp_v, ssp_hbm.at[pl.ds(base, CHA)])
        pltpu.sync_copy(rsp_v, rsp_hbm.at[pl.ds(base, CHA)])

      return carry

    lax.fori_loop(0, (NCHUNK_A + NW - 1) // NW, body, 0)

  return k(species, senders, receivers)


def _rotation_selectors():
  import numpy as np
  m = (0, 2, 6)
  rs, ss = [], []
  for j in range(3):
    r = np.zeros((81, FEAT), np.float32)
    s = np.zeros((3 * SPH, FEAT), np.float32)
    for i in range(K):
      for c in range(SPH):
        r[9 * i + m[j], i * SPH + c] = 1.0
        s[SPH * j + c, i * SPH + c] = 1.0
    rs.append(jnp.asarray(r))
    ss.append(jnp.asarray(s))
  return rs, ss


def _mlp_rotate_body(dist_ref, sp_ref, tp_ref, wig_ref,
                     r0_ref, r1_ref, r2_ref, s0_ref, s1_ref, s2_ref,
                     w0a_ref, srcw_ref, tgtw_ref, b0_ref, g0_ref, be0_ref,
                     w1_ref, b1_ref, g1_ref, be1_ref, w2_ref, b2_ref,
                     out_ref):
  def dot(a, b):
    return lax.dot_general(a, b, (((1,), (0,)), ((), ())),
                           preferred_element_type=jnp.float32)

  def ln_silu(h, g, be):
    mu = jnp.mean(h, axis=1, keepdims=True)
    d = h - mu
    var = jnp.mean(d * d, axis=1, keepdims=True)
    x = d * lax.rsqrt(var + 1e-5) * g + be
    return x / (1.0 + jnp.exp(-x))

  lanes = lax.broadcasted_iota(jnp.int32, (BLK, 128), 1)
  oh_s = (lanes == sp_ref[...]).astype(jnp.float32)
  oh_t = (lanes == tp_ref[...]).astype(jnp.float32)
  h = (dot(dist_ref[...], w0a_ref[...]) + dot(oh_s, srcw_ref[...])
       + dot(oh_t, tgtw_ref[...]) + b0_ref[...])
  h = ln_silu(h, g0_ref[...], be0_ref[...])
  h = ln_silu(dot(h, w1_ref[...]) + b1_ref[...], g1_ref[...], be1_ref[...])
  h2 = dot(h, w2_ref[...]) + b2_ref[...]

  w = wig_ref[...]
  out_ref[...] = (dot(w, r0_ref[...]) * dot(h2, s0_ref[...])
                  + dot(w, r1_ref[...]) * dot(h2, s1_ref[...])
                  + dot(w, r2_ref[...]) * dot(h2, s2_ref[...]))


def _mlp_rotate(dist, spp, tpp, wig, w0a, srcw, tgtw, b0, g0, be0,
                w1, b1, g1, be1, w2, b2):
  rs, ss = _rotation_selectors()
  edge_spec = lambda width: pl.BlockSpec((BLK, width), lambda i: (i, 0))
  full_spec = lambda a: pl.BlockSpec(a.shape, lambda i: (0, 0))
  args = (dist, spp, tpp, wig, *rs, *ss, w0a, srcw, tgtw, b0, g0, be0,
          w1, b1, g1, be1, w2, b2)
  in_specs = [edge_spec(D_EDGE), edge_spec(1), edge_spec(1),
              edge_spec(81)] + [full_spec(a) for a in args[4:]]
  return pl.pallas_call(
      _mlp_rotate_body,
      grid=(GRID,),
      in_specs=in_specs,
      out_specs=pl.BlockSpec((BLK, FEAT), lambda i: (i, 0)),
      out_shape=jax.ShapeDtypeStruct((N_EDGES, FEAT), jnp.float32),
  )(*args)


def _scatter_accumulate(receivers, rot):
  @functools.partial(
      pl.kernel,
      out_type=jax.ShapeDtypeStruct((N_NODES, FEAT), jnp.float32),
      mesh=_mesh(),
      compiler_params=pltpu.CompilerParams(
          needs_layout_passes=False, use_tc_tiling_on_sc=False),
      scratch_types=[
          pltpu.VMEM_SHARED((ACC_ROWS, FEAT), jnp.float32),
          pltpu.VMEM((16, FEAT), jnp.float32),
          pltpu.VMEM((CH,), jnp.int32),
          pltpu.VMEM((4, CH // 4), jnp.int32),
          pltpu.VMEM((2, CH // 4, FEAT), jnp.float32),
          pltpu.SemaphoreType.DMA,
          pltpu.SemaphoreType.DMA,
          pltpu.SemaphoreType.DMA,
          pltpu.SemaphoreType.DMA,
      ],
  )
  def k(recv_hbm, rot_hbm, out_hbm, acc, stage_v, ridx_v, lidx_v, feat_v,
        seml0, seml1, sems0, sems1):
    seml = (seml0, seml1)
    sems = (sems0, sems1)
    Q = CH // 4
    cid = lax.axis_index("c")
    sid = lax.axis_index("s")
    nbase = cid * HALF
    trash = HALF + sid

    zero = jnp.zeros((16,), jnp.float32)
    for r in range(16):
      for c in range(0, FEAT, 16):
        stage_v[r, pl.ds(c, 16)] = zero
    for i in range(ROWS_PER_TILE // 16):
      pltpu.sync_copy(stage_v, acc.at[pl.ds(sid * ROWS_PER_TILE + i * 16, 16)])
    plsc.subcore_barrier()

    def body(i, carry):
      chunk = i * NS + sid

      @pl.when(chunk < NCHUNK)
      def _():
        base = chunk * CH
        pltpu.sync_copy(recv_hbm.at[pl.ds(base, CH)], ridx_v)
        cmin = jnp.min(ridx_v[pl.ds(0, 16)])
        cmax = jnp.max(ridx_v[pl.ds(CH - 16, 16)])

        @pl.when((cmin < nbase + HALF) & (cmax >= nbase))
        def _():
          loads = [None] * 4
          scats = [None] * 4
          loads[0] = pltpu.async_copy(rot_hbm.at[pl.ds(base, Q)],
                                      feat_v.at[0], seml[0])
          for g in range(CH // 16):
            r16 = ridx_v[pl.ds(g * 16, 16)]
            inb = (r16 >= nbase) & (r16 < nbase + HALF)
            lidx_v[g // 2, pl.ds((g % 2) * 16, 16)] = jnp.where(
                inb, r16 - nbase, trash)
          for q in range(4):
            if q >= 1:
              scats[q - 1].wait()
            if q + 1 < 4:
              loads[q + 1] = pltpu.async_copy(
                  rot_hbm.at[pl.ds(base + (q + 1) * Q, Q)],
                  feat_v.at[(q + 1) % 2], seml[(q + 1) % 2])
            loads[q].wait()
            scats[q] = pltpu.async_copy(feat_v.at[q % 2], acc.at[lidx_v.at[q]],
                                        sems[q % 2], add=True)
          scats[3].wait()

      return carry

    lax.fori_loop(0, (NCHUNK + NS - 1) // NS, body, 0)
    plsc.subcore_barrier()

    for i in range(ROWS_PER_TILE // 16):
      start = sid * ROWS_PER_TILE + i * 16

      @pl.when(start + 16 <= HALF)
      def _():
        pltpu.sync_copy(acc.at[pl.ds(start, 16)], stage_v)
        pltpu.sync_copy(stage_v, out_hbm.at[pl.ds(nbase + start, 16)])

      @pl.when(start == HALF - 8)
      def _():
        pltpu.sync_copy(acc.at[pl.ds(start, 8)], stage_v.at[pl.ds(0, 8)])
        pltpu.sync_copy(stage_v.at[pl.ds(0, 8)],
                        out_hbm.at[pl.ds(nbase + start, 8)])

  return k(receivers, rot)


def kernel(node_species, edge_distances, senders, receivers, src_table,
           tgt_table, W0, b0, ln0_scale, ln0_bias, W1, b1, ln1_scale,
           ln1_bias, W2, b2, to_m, wigner_inv):
  species = node_species.astype(jnp.int32)
  snd = senders.astype(jnp.int32)
  rcv = receivers.astype(jnp.int32)

  spp, tpp = _gather_species(species, snd, rcv)

  w0s = W0 * (1.0 / math.sqrt(W0.shape[0]))
  w0a, w0b, w0c = w0s[:D_EDGE], w0s[D_EDGE:D_EDGE + HID], w0s[D_EDGE + HID:]
  srcw = jnp.zeros((128, HID), jnp.float32).at[:src_table.shape[0]].set(
      src_table @ w0b)
  tgtw = jnp.zeros((128, HID), jnp.float32).at[:tgt_table.shape[0]].set(
      tgt_table @ w0c)
  w1s = W1 * (1.0 / math.sqrt(W1.shape[0]))
  w2s = W2 * (1.0 / (math.sqrt(W2.shape[0]) * 16.0))
  b2s = b2 * (1.0 / 16.0)
  row = lambda v: v.reshape(1, -1)

  wig = wigner_inv.reshape(N_EDGES, 81)
  rot = _mlp_rotate(edge_distances, spp.reshape(N_EDGES, 1),
                    tpp.reshape(N_EDGES, 1), wig, w0a, srcw, tgtw,
                    row(b0), row(ln0_scale), row(ln0_bias), w1s, row(b1),
                    row(ln1_scale), row(ln1_bias), w2s, row(b2s))

  out = _scatter_accumulate(rcv, rot)
  return out.reshape(N_NODES, K, SPH)


# speedup vs baseline: 1.0944x; 1.0944x over previous
import functools
import math

import jax
import jax.numpy as jnp
from jax import lax
from jax.experimental import pallas as pl
from jax.experimental.pallas import tpu as pltpu
from jax.experimental.pallas import tpu_sc as plsc

N_NODES = 10000
N_EDGES = 160000
D_EDGE = 64
HID = 64
SPH = 32
K = 9
FEAT = K * SPH

NC = 2
NS = 16
NW = NC * NS

CH = 128
NCHUNK = N_EDGES // CH

HALF = N_NODES // 2
ACC_ROWS = 5120
ROWS_PER_TILE = ACC_ROWS // NS

BLK = 640
GRID = N_EDGES // BLK

@functools.cache
def _mesh():
  return plsc.VectorSubcoreMesh(
      core_axis_name="c", subcore_axis_name="s", num_cores=NC, num_subcores=NS
  )


CHA = 640
NCHUNK_A = N_EDGES // CHA


def _gather_species(species, senders, receivers):
  @functools.partial(
      pl.kernel,
      out_type=[
          jax.ShapeDtypeStruct((N_EDGES,), jnp.int32),
          jax.ShapeDtypeStruct((N_EDGES,), jnp.int32),
      ],
      mesh=_mesh(),
      compiler_params=pltpu.CompilerParams(
          needs_layout_passes=False, use_tc_tiling_on_sc=True),
      scratch_types=[
          pltpu.VMEM((N_NODES,), jnp.int32),
          pltpu.VMEM((CHA,), jnp.int32),
          pltpu.VMEM((CHA,), jnp.int32),
          pltpu.VMEM((CHA,), jnp.int32),
          pltpu.VMEM((CHA,), jnp.int32),
      ],
  )
  def k(species_hbm, senders_hbm, receivers_hbm, ssp_hbm, rsp_hbm,
        spec_v, sidx_v, ridx_v, ssp_v, rsp_v):
    wid = lax.axis_index("s") * NC + lax.axis_index("c")
    pltpu.sync_copy(species_hbm, spec_v)

    def body(i, carry):
      chunk = i * NW + wid

      @pl.when(chunk < NCHUNK_A)
      def _():
        base = chunk * CHA
        pltpu.sync_copy(senders_hbm.at[pl.ds(base, CHA)], sidx_v)
        pltpu.sync_copy(receivers_hbm.at[pl.ds(base, CHA)], ridx_v)
        for g in range(CHA // 16):
          sl = pl.ds(g * 16, 16)
          ssp_v[sl] = plsc.load_gather(spec_v, [sidx_v[sl]])
          rsp_v[sl] = plsc.load_gather(spec_v, [ridx_v[sl]])
        pltpu.sync_copy(ssp_v, ssp_hbm.at[pl.ds(base, CHA)])
        pltpu.sync_copy(rsp_v, rsp_hbm.at[pl.ds(base, CHA)])

      return carry

    lax.fori_loop(0, (NCHUNK_A + NW - 1) // NW, body, 0)

  return k(species, senders, receivers)


def _rotation_selectors():
  import numpy as np
  m = (0, 2, 6)
  rs, ss = [], []
  for j in range(3):
    r = np.zeros((81, FEAT), np.float32)
    s = np.zeros((3 * SPH, FEAT), np.float32)
    for i in range(K):
      for c in range(SPH):
        r[9 * i + m[j], i * SPH + c] = 1.0
        s[SPH * j + c, i * SPH + c] = 1.0
    rs.append(jnp.asarray(r))
    ss.append(jnp.asarray(s))
  return rs, ss


def _mlp_rotate_body(dist_ref, sp_ref, tp_ref, wig_ref,
                     r0_ref, r1_ref, r2_ref, s0_ref, s1_ref, s2_ref,
                     w0a_ref, srcw_ref, tgtw_ref, b0_ref, g0_ref, be0_ref,
                     w1_ref, b1_ref, g1_ref, be1_ref, w2_ref, b2_ref,
                     out_ref):
  def dot(a, b):
    return lax.dot_general(a, b, (((1,), (0,)), ((), ())),
                           preferred_element_type=jnp.float32)

  def ln_silu(h, g, be):
    mu = jnp.mean(h, axis=1, keepdims=True)
    d = h - mu
    var = jnp.mean(d * d, axis=1, keepdims=True)
    x = d * lax.rsqrt(var + 1e-5) * g + be
    return x / (1.0 + jnp.exp(-x))

  def dot_t(a, b):
    return lax.dot_general(a, b, (((0,), (0,)), ((), ())),
                           preferred_element_type=jnp.float32)

  rows = lax.broadcasted_iota(jnp.int32, (128, BLK), 0)
  oh_s = (rows == sp_ref[0]).astype(jnp.float32)
  oh_t = (rows == tp_ref[0]).astype(jnp.float32)
  h = (dot(dist_ref[...], w0a_ref[...]) + dot_t(oh_s, srcw_ref[...])
       + dot_t(oh_t, tgtw_ref[...]) + b0_ref[...])
  h = ln_silu(h, g0_ref[...], be0_ref[...])
  h = ln_silu(dot(h, w1_ref[...]) + b1_ref[...], g1_ref[...], be1_ref[...])
  h2 = dot(h, w2_ref[...]) + b2_ref[...]

  w = wig_ref[...]
  out_ref[...] = (dot(w, r0_ref[...]) * dot(h2, s0_ref[...])
                  + dot(w, r1_ref[...]) * dot(h2, s1_ref[...])
                  + dot(w, r2_ref[...]) * dot(h2, s2_ref[...]))


def _mlp_rotate(dist, spp, tpp, wig, w0a, srcw, tgtw, b0, g0, be0,
                w1, b1, g1, be1, w2, b2):
  rs, ss = _rotation_selectors()
  edge_spec = lambda width: pl.BlockSpec((BLK, width), lambda i: (i, 0))
  full_spec = lambda a: pl.BlockSpec(a.shape, lambda i: (0, 0))
  args = (dist, spp, tpp, wig, *rs, *ss, w0a, srcw, tgtw, b0, g0, be0,
          w1, b1, g1, be1, w2, b2)
  sp_spec = pl.BlockSpec((1, 1, BLK), lambda i: (i, 0, 0))
  in_specs = [edge_spec(D_EDGE), sp_spec, sp_spec,
              edge_spec(81)] + [full_spec(a) for a in args[4:]]
  return pl.pallas_call(
      _mlp_rotate_body,
      grid=(GRID,),
      in_specs=in_specs,
      out_specs=pl.BlockSpec((BLK, FEAT), lambda i: (i, 0)),
      out_shape=jax.ShapeDtypeStruct((N_EDGES, FEAT), jnp.float32),
  )(*args)


def _scatter_accumulate(receivers, rot):
  @functools.partial(
      pl.kernel,
      out_type=jax.ShapeDtypeStruct((N_NODES, FEAT), jnp.float32),
      mesh=_mesh(),
      compiler_params=pltpu.CompilerParams(
          needs_layout_passes=False, use_tc_tiling_on_sc=False),
      scratch_types=[
          pltpu.VMEM_SHARED((ACC_ROWS, FEAT), jnp.float32),
          pltpu.VMEM((16, FEAT), jnp.float32),
          pltpu.VMEM((CH,), jnp.int32),
          pltpu.VMEM((4, CH // 4), jnp.int32),
          pltpu.VMEM((2, CH // 4, FEAT), jnp.float32),
          pltpu.SemaphoreType.DMA,
          pltpu.SemaphoreType.DMA,
          pltpu.SemaphoreType.DMA,
          pltpu.SemaphoreType.DMA,
      ],
  )
  def k(recv_hbm, rot_hbm, out_hbm, acc, stage_v, ridx_v, lidx_v, feat_v,
        seml0, seml1, sems0, sems1):
    seml = (seml0, seml1)
    sems = (sems0, sems1)
    Q = CH // 4
    cid = lax.axis_index("c")
    sid = lax.axis_index("s")
    nbase = cid * HALF
    trash = HALF + sid

    zero = jnp.zeros((16,), jnp.float32)
    for r in range(16):
      for c in range(0, FEAT, 16):
        stage_v[r, pl.ds(c, 16)] = zero
    for i in range(ROWS_PER_TILE // 16):
      pltpu.sync_copy(stage_v, acc.at[pl.ds(sid * ROWS_PER_TILE + i * 16, 16)])
    plsc.subcore_barrier()

    def body(i, carry):
      chunk = i * NS + sid

      @pl.when(chunk < NCHUNK)
      def _():
        base = chunk * CH
        pltpu.sync_copy(recv_hbm.at[pl.ds(base, CH)], ridx_v)
        cmin = jnp.min(ridx_v[pl.ds(0, 16)])
        cmax = jnp.max(ridx_v[pl.ds(CH - 16, 16)])

        @pl.when((cmin < nbase + HALF) & (cmax >= nbase))
        def _():
          loads = [None] * 4
          scats = [None] * 4
          loads[0] = pltpu.async_copy(rot_hbm.at[pl.ds(base, Q)],
                                      feat_v.at[0], seml[0])
          for g in range(CH // 16):
            r16 = ridx_v[pl.ds(g * 16, 16)]
            inb = (r16 >= nbase) & (r16 < nbase + HALF)
            lidx_v[g // 2, pl.ds((g % 2) * 16, 16)] = jnp.where(
                inb, r16 - nbase, trash)
          for q in range(4):
            if q >= 1:
              scats[q - 1].wait()
            if q + 1 < 4:
              loads[q + 1] = pltpu.async_copy(
                  rot_hbm.at[pl.ds(base + (q + 1) * Q, Q)],
                  feat_v.at[(q + 1) % 2], seml[(q + 1) % 2])
            loads[q].wait()
            scats[q] = pltpu.async_copy(feat_v.at[q % 2], acc.at[lidx_v.at[q]],
                                        sems[q % 2], add=True)
          scats[3].wait()

      return carry

    lax.fori_loop(0, (NCHUNK + NS - 1) // NS, body, 0)
    plsc.subcore_barrier()

    for i in range(ROWS_PER_TILE // 16):
      start = sid * ROWS_PER_TILE + i * 16

      @pl.when(start + 16 <= HALF)
      def _():
        pltpu.sync_copy(acc.at[pl.ds(start, 16)], stage_v)
        pltpu.sync_copy(stage_v, out_hbm.at[pl.ds(nbase + start, 16)])

      @pl.when(start == HALF - 8)
      def _():
        pltpu.sync_copy(acc.at[pl.ds(start, 8)], stage_v.at[pl.ds(0, 8)])
        pltpu.sync_copy(stage_v.at[pl.ds(0, 8)],
                        out_hbm.at[pl.ds(nbase + start, 8)])

  return k(receivers, rot)


def kernel(node_species, edge_distances, senders, receivers, src_table,
           tgt_table, W0, b0, ln0_scale, ln0_bias, W1, b1, ln1_scale,
           ln1_bias, W2, b2, to_m, wigner_inv):
  species = node_species.astype(jnp.int32)
  snd = senders.astype(jnp.int32)
  rcv = receivers.astype(jnp.int32)

  spp, tpp = _gather_species(species, snd, rcv)

  w0s = W0 * (1.0 / math.sqrt(W0.shape[0]))
  w0a, w0b, w0c = w0s[:D_EDGE], w0s[D_EDGE:D_EDGE + HID], w0s[D_EDGE + HID:]
  srcw = jnp.zeros((128, HID), jnp.float32).at[:src_table.shape[0]].set(
      src_table @ w0b)
  tgtw = jnp.zeros((128, HID), jnp.float32).at[:tgt_table.shape[0]].set(
      tgt_table @ w0c)
  w1s = W1 * (1.0 / math.sqrt(W1.shape[0]))
  w2s = W2 * (1.0 / (math.sqrt(W2.shape[0]) * 16.0))
  b2s = b2 * (1.0 / 16.0)
  row = lambda v: v.reshape(1, -1)

  wig = wigner_inv.reshape(N_EDGES, 81)
  rot = _mlp_rotate(edge_distances, spp.reshape(GRID, 1, BLK),
                    tpp.reshape(GRID, 1, BLK), wig, w0a, srcw, tgtw,
                    row(b0), row(ln0_scale), row(ln0_bias), w1s, row(b1),
                    row(ln1_scale), row(ln1_bias), w2s, row(b2s))

  out = _scatter_accumulate(rcv, rot)
  return out.reshape(N_NODES, K, SPH)
